# baseline (speedup 1.0000x reference)
import functools

import jax
import jax.numpy as jnp
from jax import lax
from jax.experimental import pallas as pl
from jax.experimental.pallas import tpu as pltpu
from jax.experimental.pallas import tpu_sc as plsc

NC = 2
NS = 16
NT = NC * NS

N = 10000
NP = 10240
E = 160000
G = 64
D_IN = 256
H = 128
C_OUT = 2

EDGE_CHUNK = 128
CH = 40
EPT = CH * EDGE_CHUNK
EP = NT * EPT
ROWS_PER_TILE = NP // NS


def _sc_mesh():
    return plsc.VectorSubcoreMesh(core_axis_name="c", subcore_axis_name="s",
                                  num_cores=NC, num_subcores=NS)


def _deg_kernel(dst_hbm, out_hbm, idx_v, cnt_v):
    c = lax.axis_index("c")
    s = lax.axis_index("s")
    wid = c * NS + s

    zeros = jnp.zeros((16,), jnp.float32)

    def zero_body(i, _):
        cnt_v[pl.ds(i * 16, 16)] = zeros
        return 0

    lax.fori_loop(0, NP // 16, zero_body, 0)

    pltpu.sync_copy(dst_hbm.at[pl.ds(wid * EPT, EPT)], idx_v)

    ones = jnp.ones((16,), jnp.float32)

    def body(i, _):
        idx = idx_v[pl.ds(i * 16, 16)]
        plsc.addupdate_scatter(cnt_v, [idx], ones)
        return 0

    lax.fori_loop(0, EPT // 16, body, 0)
    pltpu.sync_copy(cnt_v, out_hbm.at[wid])


def _deg_call(dst_flat):
    return pl.kernel(
        _deg_kernel,
        out_type=jax.ShapeDtypeStruct((NT, NP), jnp.float32),
        mesh=_sc_mesh(),
        compiler_params=pltpu.CompilerParams(needs_layout_passes=False),
        scratch_types=[
            pltpu.VMEM((EPT,), jnp.int32),
            pltpu.VMEM((NP,), jnp.float32),
        ],
    )(dst_flat)


def _scatter_kernel(g_hbm, src_hbm, dst_hbm, out_hbm,
                    src_v, dst_v, rows_a, rows_b, zero_v, acc_sh,
                    sem_a, sem_b):
    c = lax.axis_index("c")
    s = lax.axis_index("s")
    wid = (1 - c) * NS + s
    row0 = s * ROWS_PER_TILE

    zeros = jnp.zeros((16,), jnp.float32)

    with jax.named_scope("sc_zero"):
        def zbuf_body(i, _):
            for j in range(H // 16):
                zero_v[i, pl.ds(j * 16, 16)] = zeros
            return 0

        lax.fori_loop(0, 16, zbuf_body, 0)

        def zacc_body(i, _):
            pltpu.sync_copy(zero_v, acc_sh.at[pl.ds(row0 + i * 16, 16)])
            return 0

        lax.fori_loop(0, ROWS_PER_TILE // 16, zacc_body, 0)

    with jax.named_scope("sc_idx"):
        pltpu.sync_copy(src_hbm.at[pl.ds(wid * CH, CH)], src_v)
        pltpu.sync_copy(dst_hbm.at[pl.ds(wid * CH, CH)], dst_v)
        plsc.subcore_barrier()

    with jax.named_scope("sc_edges"):
        pltpu.async_copy(g_hbm.at[src_v.at[0]], rows_a, sem_a)

        def body(t, _):
            j = 2 * t
            pltpu.async_copy(g_hbm.at[src_v.at[j + 1]], rows_b, sem_b)
            pltpu.make_async_copy(g_hbm.at[src_v.at[j]], rows_a, sem_a).wait()
            pltpu.sync_copy(rows_a, acc_sh.at[dst_v.at[j]], add=True)

            @pl.when(t + 1 < CH // 2)
            def _():
                pltpu.async_copy(g_hbm.at[src_v.at[j + 2]], rows_a, sem_a)

            pltpu.make_async_copy(g_hbm.at[src_v.at[j + 1]], rows_b,
                                  sem_b).wait()
            pltpu.sync_copy(rows_b, acc_sh.at[dst_v.at[j + 1]], add=True)
            return 0

        lax.fori_loop(0, CH // 2, body, 0)
        plsc.subcore_barrier()

    with jax.named_scope("sc_writeback"):
        pltpu.sync_copy(acc_sh.at[pl.ds(row0, ROWS_PER_TILE)],
                        out_hbm.at[c, pl.ds(row0, ROWS_PER_TILE)])


def _scatter_call(g_pad, src2d, dst2d):
    return pl.kernel(
        _scatter_kernel,
        out_type=jax.ShapeDtypeStruct((NC, NP, H), jnp.float32),
        mesh=_sc_mesh(),
        compiler_params=pltpu.CompilerParams(needs_layout_passes=False),
        scratch_types=[
            pltpu.VMEM((CH, EDGE_CHUNK), jnp.int32),
            pltpu.VMEM((CH, EDGE_CHUNK), jnp.int32),
            pltpu.VMEM((EDGE_CHUNK, H), jnp.float32),
            pltpu.VMEM((EDGE_CHUNK, H), jnp.float32),
            pltpu.VMEM((16, H), jnp.float32),
            pltpu.VMEM_SHARED((NP, H), jnp.float32),
            pltpu.SemaphoreType.DMA,
            pltpu.SemaphoreType.DMA,
        ],
    )(g_pad, src2d, dst2d)


def _g_kernel(x_ref, w_ref, deg_ref, g_ref, dinv_ref):
    h = jnp.dot(x_ref[...], w_ref[...], preferred_element_type=jnp.float32)
    cnt = jnp.sum(deg_ref[...], axis=1, keepdims=True)
    dinv = lax.rsqrt(cnt + 1.0)
    g_ref[...] = h * dinv
    dinv_ref[...] = dinv


def _g_call(x_pad, w_conv, deg_t):
    blk = 512
    grid = NP // blk
    return pl.pallas_call(
        _g_kernel,
        grid=(grid,),
        in_specs=[
            pl.BlockSpec((blk, D_IN), lambda i: (i, 0)),
            pl.BlockSpec((D_IN, H), lambda i: (0, 0)),
            pl.BlockSpec((blk, NT), lambda i: (i, 0)),
        ],
        out_specs=[
            pl.BlockSpec((blk, H), lambda i: (i, 0)),
            pl.BlockSpec((blk, 1), lambda i: (i, 0)),
        ],
        out_shape=[
            jax.ShapeDtypeStruct((NP, H), jnp.float32),
            jax.ShapeDtypeStruct((NP, 1), jnp.float32),
        ],
    )(x_pad, w_conv, deg_t)


_DB = 2048


def _pool_kernel(s0_ref, s1_ref, g_ref, dinv_ref, bconv_ref, batch_ref,
                 batch_s_ref, hp_ref, root_ref,
                 hsum_ref, hmax_ref, rootsum_ref, carry_ref):
    bi = pl.program_id(0)
    nb = pl.num_programs(0)

    @pl.when(bi == 0)
    def _init():
        hsum_ref[...] = jnp.zeros((G, H), jnp.float32)
        hmax_ref[...] = jnp.full((G, H), -jnp.inf, jnp.float32)
        rootsum_ref[...] = jnp.zeros((G,), jnp.int32)
        carry_ref[0] = batch_s_ref[0]
        carry_ref[1] = 0

    hc = (s0_ref[...] + s1_ref[...] + g_ref[...]) * dinv_ref[...]
    hc = jnp.maximum(hc + bconv_ref[...][None, :], 0.0)

    bb = batch_ref[...]
    ridx = lax.broadcasted_iota(jnp.int32, (_DB, 1), 0) + bi * _DB
    valid = ridx < N

    prev_last = carry_ref[0]
    cnt_carry = carry_ref[1]
    shifted = jnp.concatenate(
        [jnp.full((1, 1), prev_last, jnp.int32), bb[:-1]], axis=0)
    bvec = jnp.where((bb != shifted) & valid, 1, 0)

    k = bvec
    sh = 1
    while sh < _DB:
        k = k + jnp.concatenate(
            [jnp.zeros((sh, 1), jnp.int32), k[:-sh]], axis=0)
        sh *= 2
    k = k + cnt_carry

    kb = jnp.broadcast_to(k, (_DB, G))
    rows = lax.broadcasted_iota(jnp.int32, (_DB, G), 1)
    hit = (kb == rows) & jnp.broadcast_to(bvec == 1, (_DB, G))
    contrib = jnp.where(hit, jnp.broadcast_to(ridx, (_DB, G)), 0)
    rootsum_ref[...] += jnp.sum(contrib, axis=0)

    carry_ref[0] = batch_s_ref[_DB - 1]
    carry_ref[1] = cnt_carry + jnp.sum(bvec)

    gmin = batch_s_ref[0]
    gmax = batch_s_ref[_DB - 1]

    def gbody(gi, _):
        mask = (bb == gi) & valid
        ssum = jnp.sum(jnp.where(mask, hc, 0.0), axis=0, keepdims=True)
        smax = jnp.max(jnp.where(mask, hc, -jnp.inf), axis=0, keepdims=True)
        hsum_ref[pl.ds(gi, 1), :] += ssum
        hmax_ref[pl.ds(gi, 1), :] = jnp.maximum(hmax_ref[pl.ds(gi, 1), :],
                                                smax)
        return 0

    lax.fori_loop(gmin, gmax + 1, gbody, 0)

    @pl.when(bi == nb - 1)
    def _fin():
        hp_ref[...] = hmax_ref[...] + hsum_ref[...]
        iota_g = lax.broadcasted_iota(jnp.int32, (G,), 0)
        root_ref[...] = jnp.where(iota_g == 0, 0,
                                  jnp.maximum(rootsum_ref[...], 1))


def _pool_call(s0, s1, g_pad, dinv, b_conv, batch2d, batch_pad):
    grid = NP // _DB
    return pl.pallas_call(
        _pool_kernel,
        grid=(grid,),
        in_specs=[
            pl.BlockSpec((_DB, H), lambda i: (i, 0)),
            pl.BlockSpec((_DB, H), lambda i: (i, 0)),
            pl.BlockSpec((_DB, H), lambda i: (i, 0)),
            pl.BlockSpec((_DB, 1), lambda i: (i, 0)),
            pl.BlockSpec((H,), lambda i: (0,)),
            pl.BlockSpec((_DB, 1), lambda i: (i, 0)),
            pl.BlockSpec((_DB,), lambda i: (i,), memory_space=pltpu.SMEM),
        ],
        out_specs=[
            pl.BlockSpec((G, H), lambda i: (0, 0)),
            pl.BlockSpec((G,), lambda i: (0,)),
        ],
        out_shape=[
            jax.ShapeDtypeStruct((G, H), jnp.float32),
            jax.ShapeDtypeStruct((G,), jnp.int32),
        ],
        scratch_shapes=[
            pltpu.VMEM((G, H), jnp.float32),
            pltpu.VMEM((G, H), jnp.float32),
            pltpu.VMEM((G,), jnp.int32),
            pltpu.SMEM((2,), jnp.int32),
        ],
    )(s0, s1, g_pad, dinv, b_conv, batch2d, batch_pad)


def _gather_kernel(root_ref, x_ref, o_ref):
    o_ref[...] = x_ref[...]


def _gather_call(root, x):
    x3 = x.reshape(N, 2, 128)
    grid_spec = pltpu.PrefetchScalarGridSpec(
        num_scalar_prefetch=1,
        grid=(G,),
        in_specs=[pl.BlockSpec((1, 2, 128), lambda i, root: (root[i], 0, 0))],
        out_specs=pl.BlockSpec((1, 2, 128), lambda i, root: (i, 0, 0)),
    )
    out = pl.pallas_call(
        _gather_kernel,
        grid_spec=grid_spec,
        out_shape=jax.ShapeDtypeStruct((G, 2, 128), jnp.float32),
    )(root, x3)
    return out.reshape(G, D_IN)


def _head_kernel(ni_ref, hp_ref, w0_ref, b0_ref, w1_ref, b1_ref,
                 w2_ref, b2_ref, o_ref):
    news = jnp.dot(ni_ref[...], w0_ref[...],
                   preferred_element_type=jnp.float32)
    news = jnp.maximum(news + b0_ref[...][None, :], 0.0)
    cat = jnp.concatenate([news, hp_ref[...]], axis=1)
    h2 = jnp.dot(cat, w1_ref[...], preferred_element_type=jnp.float32)
    h2 = jnp.maximum(h2 + b1_ref[...][None, :], 0.0)
    logits = jnp.dot(h2, w2_ref[...], preferred_element_type=jnp.float32)
    logits = logits + b2_ref[...][None, :]
    mx = jnp.max(logits, axis=1, keepdims=True)
    lse = jnp.log(jnp.sum(jnp.exp(logits - mx), axis=1, keepdims=True)) + mx
    o_ref[...] = logits - lse


def _head_call(news_in, hp, w0, b0, w1, b1, w2, b2):
    return pl.pallas_call(
        _head_kernel,
        out_shape=jax.ShapeDtypeStruct((G, C_OUT), jnp.float32),
    )(news_in, hp, w0, b0, w1, b1, w2, b2)


def kernel(x, edge_index, batch, W_conv, b_conv, W0, b0, W1, b1, W2, b2):
    src = edge_index[0]
    dst = edge_index[1]
    pad = EP - E
    src_p = jnp.concatenate([src, jnp.zeros((pad,), jnp.int32)])
    pad_dst = N + (jnp.arange(pad, dtype=jnp.int32) % (NP - N))
    dst_p = jnp.concatenate([dst, pad_dst])
    src2d = src_p.reshape(EP // EDGE_CHUNK, EDGE_CHUNK)
    dst2d = dst_p.reshape(EP // EDGE_CHUNK, EDGE_CHUNK)

    x_pad = jnp.concatenate(
        [x, jnp.zeros((NP - N, D_IN), jnp.float32)], axis=0)
    batch_pad = jnp.concatenate(
        [batch, jnp.full((NP - N,), G - 1, jnp.int32)])

    deg_parts = _deg_call(dst_p)
    deg_t = deg_parts.T
    g_pad, dinv = _g_call(x_pad, W_conv, deg_t)
    s_parts = _scatter_call(g_pad, src2d, dst2d)
    hp, root = _pool_call(s_parts[0], s_parts[1], g_pad, dinv,
                          b_conv, batch_pad[:, None], batch_pad)
    news_in = _gather_call(root, x)
    return _head_call(news_in, hp, W0, b0, W1, b1, W2, b2)


# speedup vs baseline: 1.9784x; 1.9784x over previous
import functools

import jax
import jax.numpy as jnp
from jax import lax
from jax.experimental import pallas as pl
from jax.experimental.pallas import tpu as pltpu
from jax.experimental.pallas import tpu_sc as plsc

NC = 2
NS = 16
NT = NC * NS

N = 10000
NP = 10240
E = 160000
G = 64
D_IN = 256
H = 128
C_OUT = 2

EDGE_CHUNK = 128
CH = 40
EPT = CH * EDGE_CHUNK
EP = NT * EPT
ROWS_PER_TILE = NP // NS


def _sc_mesh():
    return plsc.VectorSubcoreMesh(core_axis_name="c", subcore_axis_name="s",
                                  num_cores=NC, num_subcores=NS)


def _deg_kernel(dst_hbm, out_hbm, idx_v, cnt_v):
    c = lax.axis_index("c")
    s = lax.axis_index("s")
    wid = c * NS + s

    zeros = jnp.zeros((16,), jnp.float32)

    def zero_body(i, _):
        cnt_v[pl.ds(i * 16, 16)] = zeros
        return 0

    lax.fori_loop(0, NP // 16, zero_body, 0)

    pltpu.sync_copy(dst_hbm.at[pl.ds(wid * EPT, EPT)], idx_v)

    ones = jnp.ones((16,), jnp.float32)

    def body(i, _):
        idx = idx_v[pl.ds(i * 16, 16)]
        plsc.addupdate_scatter(cnt_v, [idx], ones)
        return 0

    lax.fori_loop(0, EPT // 16, body, 0)
    pltpu.sync_copy(cnt_v, out_hbm.at[wid])


def _deg_call(dst_flat):
    return pl.kernel(
        _deg_kernel,
        out_type=jax.ShapeDtypeStruct((NT, NP), jnp.float32),
        mesh=_sc_mesh(),
        compiler_params=pltpu.CompilerParams(needs_layout_passes=False),
        scratch_types=[
            pltpu.VMEM((EPT,), jnp.int32),
            pltpu.VMEM((NP,), jnp.float32),
        ],
    )(dst_flat)


def _scatter_kernel(g_hbm, src_hbm, dst_hbm, out_hbm,
                    src_v, dst_v, rows_a, rows_b, zero_v, acc_sh,
                    sem_a, sem_b):
    c = lax.axis_index("c")
    s = lax.axis_index("s")
    wid = c * NS + s
    row0 = s * ROWS_PER_TILE

    zeros = jnp.zeros((16,), jnp.float32)

    with jax.named_scope("sc_zero"):
        def zbuf_body(i, _):
            for j in range(H // 16):
                zero_v[i, pl.ds(j * 16, 16)] = zeros
            return 0

        lax.fori_loop(0, 16, zbuf_body, 0)

        def zacc_body(i, _):
            pltpu.sync_copy(zero_v, acc_sh.at[pl.ds(row0 + i * 16, 16)])
            return 0

        lax.fori_loop(0, ROWS_PER_TILE // 16, zacc_body, 0)

    with jax.named_scope("sc_idx"):
        pltpu.sync_copy(src_hbm.at[pl.ds(wid * CH, CH)], src_v)
        pltpu.sync_copy(dst_hbm.at[pl.ds(wid * CH, CH)], dst_v)
        plsc.subcore_barrier()

    with jax.named_scope("sc_edges"):
        pltpu.async_copy(g_hbm.at[src_v.at[0]], rows_a, sem_a)

        def body(t, _):
            j = 2 * t
            pltpu.async_copy(g_hbm.at[src_v.at[j + 1]], rows_b, sem_b)
            pltpu.make_async_copy(g_hbm.at[src_v.at[j]], rows_a, sem_a).wait()
            pltpu.sync_copy(rows_a, acc_sh.at[dst_v.at[j]], add=True)

            @pl.when(t + 1 < CH // 2)
            def _():
                pltpu.async_copy(g_hbm.at[src_v.at[j + 2]], rows_a, sem_a)

            pltpu.make_async_copy(g_hbm.at[src_v.at[j + 1]], rows_b,
                                  sem_b).wait()
            pltpu.sync_copy(rows_b, acc_sh.at[dst_v.at[j + 1]], add=True)
            return 0

        lax.fori_loop(0, CH // 2, body, 0)
        plsc.subcore_barrier()

    with jax.named_scope("sc_writeback"):
        pltpu.sync_copy(acc_sh.at[pl.ds(row0, ROWS_PER_TILE)],
                        out_hbm.at[c, pl.ds(row0, ROWS_PER_TILE)])


def _scatter_call(g_pad, src2d, dst2d):
    return pl.kernel(
        _scatter_kernel,
        out_type=jax.ShapeDtypeStruct((NC, NP, H), jnp.float32),
        mesh=_sc_mesh(),
        compiler_params=pltpu.CompilerParams(needs_layout_passes=False),
        scratch_types=[
            pltpu.VMEM((CH, EDGE_CHUNK), jnp.int32),
            pltpu.VMEM((CH, EDGE_CHUNK), jnp.int32),
            pltpu.VMEM((EDGE_CHUNK, H), jnp.float32),
            pltpu.VMEM((EDGE_CHUNK, H), jnp.float32),
            pltpu.VMEM((16, H), jnp.float32),
            pltpu.VMEM_SHARED((NP, H), jnp.float32),
            pltpu.SemaphoreType.DMA,
            pltpu.SemaphoreType.DMA,
        ],
    )(g_pad, src2d, dst2d)


def _g_kernel(x_ref, w_ref, deg_ref, g_ref, dinv_ref):
    h = jnp.dot(x_ref[...], w_ref[...], preferred_element_type=jnp.float32)
    cnt = jnp.sum(deg_ref[...], axis=1, keepdims=True)
    dinv = lax.rsqrt(cnt + 1.0)
    g_ref[...] = h * dinv
    dinv_ref[...] = dinv


def _g_call(x_pad, w_conv, deg_t):
    blk = 512
    grid = NP // blk
    return pl.pallas_call(
        _g_kernel,
        grid=(grid,),
        in_specs=[
            pl.BlockSpec((blk, D_IN), lambda i: (i, 0)),
            pl.BlockSpec((D_IN, H), lambda i: (0, 0)),
            pl.BlockSpec((blk, NT), lambda i: (i, 0)),
        ],
        out_specs=[
            pl.BlockSpec((blk, H), lambda i: (i, 0)),
            pl.BlockSpec((blk, 1), lambda i: (i, 0)),
        ],
        out_shape=[
            jax.ShapeDtypeStruct((NP, H), jnp.float32),
            jax.ShapeDtypeStruct((NP, 1), jnp.float32),
        ],
    )(x_pad, w_conv, deg_t)


_DB = 2048


def _pool_kernel(s0_ref, s1_ref, g_ref, dinv_ref, bconv_ref, batch_ref,
                 batch_s_ref, hp_ref, root_ref,
                 hsum_ref, hmax_ref, rootsum_ref, carry_ref):
    bi = pl.program_id(0)
    nb = pl.num_programs(0)

    @pl.when(bi == 0)
    def _init():
        hsum_ref[...] = jnp.zeros((G, H), jnp.float32)
        hmax_ref[...] = jnp.full((G, H), -jnp.inf, jnp.float32)
        rootsum_ref[...] = jnp.zeros((G,), jnp.int32)
        carry_ref[0] = batch_s_ref[0]
        carry_ref[1] = 0

    hc = (s0_ref[...] + s1_ref[...] + g_ref[...]) * dinv_ref[...]
    hc = jnp.maximum(hc + bconv_ref[...][None, :], 0.0)

    bb = batch_ref[...]
    ridx = lax.broadcasted_iota(jnp.int32, (_DB, 1), 0) + bi * _DB
    valid = ridx < N

    prev_last = carry_ref[0]
    cnt_carry = carry_ref[1]
    shifted = jnp.concatenate(
        [jnp.full((1, 1), prev_last, jnp.int32), bb[:-1]], axis=0)
    bvec = jnp.where((bb != shifted) & valid, 1, 0)

    k = bvec
    sh = 1
    while sh < _DB:
        k = k + jnp.concatenate(
            [jnp.zeros((sh, 1), jnp.int32), k[:-sh]], axis=0)
        sh *= 2
    k = k + cnt_carry

    kb = jnp.broadcast_to(k, (_DB, G))
    rows = lax.broadcasted_iota(jnp.int32, (_DB, G), 1)
    hit = (kb == rows) & jnp.broadcast_to(bvec == 1, (_DB, G))
    contrib = jnp.where(hit, jnp.broadcast_to(ridx, (_DB, G)), 0)
    rootsum_ref[...] += jnp.sum(contrib, axis=0)

    carry_ref[0] = batch_s_ref[_DB - 1]
    carry_ref[1] = cnt_carry + jnp.sum(bvec)

    gmin = batch_s_ref[0]
    gmax = batch_s_ref[_DB - 1]

    def gbody(gi, _):
        mask = (bb == gi) & valid
        ssum = jnp.sum(jnp.where(mask, hc, 0.0), axis=0, keepdims=True)
        smax = jnp.max(jnp.where(mask, hc, -jnp.inf), axis=0, keepdims=True)
        hsum_ref[pl.ds(gi, 1), :] += ssum
        hmax_ref[pl.ds(gi, 1), :] = jnp.maximum(hmax_ref[pl.ds(gi, 1), :],
                                                smax)
        return 0

    lax.fori_loop(gmin, gmax + 1, gbody, 0)

    @pl.when(bi == nb - 1)
    def _fin():
        hp_ref[...] = hmax_ref[...] + hsum_ref[...]
        iota_g = lax.broadcasted_iota(jnp.int32, (G,), 0)
        root_ref[...] = jnp.where(iota_g == 0, 0,
                                  jnp.maximum(rootsum_ref[...], 1))


def _pool_call(s0, s1, g_pad, dinv, b_conv, batch2d, batch_pad):
    grid = NP // _DB
    return pl.pallas_call(
        _pool_kernel,
        grid=(grid,),
        in_specs=[
            pl.BlockSpec((_DB, H), lambda i: (i, 0)),
            pl.BlockSpec((_DB, H), lambda i: (i, 0)),
            pl.BlockSpec((_DB, H), lambda i: (i, 0)),
            pl.BlockSpec((_DB, 1), lambda i: (i, 0)),
            pl.BlockSpec((H,), lambda i: (0,)),
            pl.BlockSpec((_DB, 1), lambda i: (i, 0)),
            pl.BlockSpec((_DB,), lambda i: (i,), memory_space=pltpu.SMEM),
        ],
        out_specs=[
            pl.BlockSpec((G, H), lambda i: (0, 0)),
            pl.BlockSpec((G,), lambda i: (0,)),
        ],
        out_shape=[
            jax.ShapeDtypeStruct((G, H), jnp.float32),
            jax.ShapeDtypeStruct((G,), jnp.int32),
        ],
        scratch_shapes=[
            pltpu.VMEM((G, H), jnp.float32),
            pltpu.VMEM((G, H), jnp.float32),
            pltpu.VMEM((G,), jnp.int32),
            pltpu.SMEM((2,), jnp.int32),
        ],
    )(s0, s1, g_pad, dinv, b_conv, batch2d, batch_pad)


def _gather_kernel(root_ref, x_ref, o_ref):
    o_ref[...] = x_ref[...]


def _gather_call(root, x):
    x3 = x.reshape(N, 2, 128)
    grid_spec = pltpu.PrefetchScalarGridSpec(
        num_scalar_prefetch=1,
        grid=(G,),
        in_specs=[pl.BlockSpec((1, 2, 128), lambda i, root: (root[i], 0, 0))],
        out_specs=pl.BlockSpec((1, 2, 128), lambda i, root: (i, 0, 0)),
    )
    out = pl.pallas_call(
        _gather_kernel,
        grid_spec=grid_spec,
        out_shape=jax.ShapeDtypeStruct((G, 2, 128), jnp.float32),
    )(root, x3)
    return out.reshape(G, D_IN)


def _head_kernel(ni_ref, hp_ref, w0_ref, b0_ref, w1_ref, b1_ref,
                 w2_ref, b2_ref, o_ref):
    news = jnp.dot(ni_ref[...], w0_ref[...],
                   preferred_element_type=jnp.float32)
    news = jnp.maximum(news + b0_ref[...][None, :], 0.0)
    cat = jnp.concatenate([news, hp_ref[...]], axis=1)
    h2 = jnp.dot(cat, w1_ref[...], preferred_element_type=jnp.float32)
    h2 = jnp.maximum(h2 + b1_ref[...][None, :], 0.0)
    logits = jnp.dot(h2, w2_ref[...], preferred_element_type=jnp.float32)
    logits = logits + b2_ref[...][None, :]
    mx = jnp.max(logits, axis=1, keepdims=True)
    lse = jnp.log(jnp.sum(jnp.exp(logits - mx), axis=1, keepdims=True)) + mx
    o_ref[...] = logits - lse


def _head_call(news_in, hp, w0, b0, w1, b1, w2, b2):
    return pl.pallas_call(
        _head_kernel,
        out_shape=jax.ShapeDtypeStruct((G, C_OUT), jnp.float32),
    )(news_in, hp, w0, b0, w1, b1, w2, b2)


def kernel(x, edge_index, batch, W_conv, b_conv, W0, b0, W1, b1, W2, b2):
    src = edge_index[0]
    dst = edge_index[1]
    pad = EP - E
    pad_rows = N + (jnp.arange(pad, dtype=jnp.int32) % (NP - N))
    src_p = jnp.concatenate([src, pad_rows])
    dst_p = jnp.concatenate([dst, pad_rows])
    src2d = src_p.reshape(EP // EDGE_CHUNK, EDGE_CHUNK)
    dst2d = dst_p.reshape(EP // EDGE_CHUNK, EDGE_CHUNK)

    x_pad = jnp.concatenate(
        [x, jnp.zeros((NP - N, D_IN), jnp.float32)], axis=0)
    batch_pad = jnp.concatenate(
        [batch, jnp.full((NP - N,), G - 1, jnp.int32)])

    deg_parts = _deg_call(dst_p)
    deg_t = deg_parts.T
    g_pad, dinv = _g_call(x_pad, W_conv, deg_t)
    s_parts = _scatter_call(g_pad, src2d, dst2d)
    hp, root = _pool_call(s_parts[0], s_parts[1], g_pad, dinv,
                          b_conv, batch_pad[:, None], batch_pad)
    news_in = _gather_call(root, x)
    return _head_call(news_in, hp, W0, b0, W1, b1, W2, b2)


# speedup vs baseline: 2.6899x; 1.3596x over previous
import functools

import jax
import jax.numpy as jnp
from jax import lax
from jax.experimental import pallas as pl
from jax.experimental.pallas import tpu as pltpu
from jax.experimental.pallas import tpu_sc as plsc

NC = 2
NS = 16
NT = NC * NS

N = 10000
NP = 10240
E = 160000
G = 64
D_IN = 256
H = 128
C_OUT = 2

EDGE_CHUNK = 128
CH = 40
EPT = CH * EDGE_CHUNK
EP = NT * EPT
ROWS_PER_TILE = NP // NS


def _sc_mesh():
    return plsc.VectorSubcoreMesh(core_axis_name="c", subcore_axis_name="s",
                                  num_cores=NC, num_subcores=NS)


def _deg_kernel(dst_hbm, out_hbm, idx_v, cnt_v):
    c = lax.axis_index("c")
    s = lax.axis_index("s")
    wid = c * NS + s

    zeros = jnp.zeros((16,), jnp.float32)

    def zero_body(i, _):
        cnt_v[pl.ds(i * 16, 16)] = zeros
        return 0

    lax.fori_loop(0, NP // 16, zero_body, 0)

    pltpu.sync_copy(dst_hbm.at[pl.ds(wid * EPT, EPT)], idx_v)

    ones = jnp.ones((16,), jnp.float32)

    def body(i, _):
        idx = idx_v[pl.ds(i * 16, 16)]
        plsc.addupdate_scatter(cnt_v, [idx], ones)
        return 0

    lax.fori_loop(0, EPT // 16, body, 0)
    pltpu.sync_copy(cnt_v, out_hbm.at[wid])


def _deg_call(dst_flat):
    return pl.kernel(
        _deg_kernel,
        out_type=jax.ShapeDtypeStruct((NT, NP), jnp.float32),
        mesh=_sc_mesh(),
        compiler_params=pltpu.CompilerParams(needs_layout_passes=False),
        scratch_types=[
            pltpu.VMEM((EPT,), jnp.int32),
            pltpu.VMEM((NP,), jnp.float32),
        ],
    )(dst_flat)


def _scatter_kernel(g_hbm, src_hbm, dst_hbm, out_hbm,
                    src_v, dst_v, rows_a, rows_b, zero_v, acc_sh,
                    sem_a, sem_b):
    c = lax.axis_index("c")
    s = lax.axis_index("s")
    wid = c * NS + s
    row0 = s * ROWS_PER_TILE

    zeros = jnp.zeros((16,), jnp.float32)

    with jax.named_scope("sc_zero"):
        def zbuf_body(i, _):
            for j in range(H // 16):
                zero_v[i, pl.ds(j * 16, 16)] = zeros
            return 0

        lax.fori_loop(0, 16, zbuf_body, 0)

        def zacc_body(i, _):
            pltpu.sync_copy(zero_v, acc_sh.at[pl.ds(row0 + i * 16, 16)])
            return 0

        lax.fori_loop(0, ROWS_PER_TILE // 16, zacc_body, 0)

    with jax.named_scope("sc_idx"):
        pltpu.sync_copy(src_hbm.at[pl.ds(wid * CH, CH)], src_v)
        pltpu.sync_copy(dst_hbm.at[pl.ds(wid * CH, CH)], dst_v)
        plsc.subcore_barrier()

    with jax.named_scope("sc_edges"):
        pltpu.async_copy(g_hbm.at[src_v.at[0]], rows_a, sem_a)

        def body(t, _):
            j = 2 * t
            pltpu.async_copy(g_hbm.at[src_v.at[j + 1]], rows_b, sem_b)
            pltpu.make_async_copy(g_hbm.at[src_v.at[j]], rows_a, sem_a).wait()
            pltpu.sync_copy(rows_a, acc_sh.at[dst_v.at[j]], add=True)

            @pl.when(t + 1 < CH // 2)
            def _():
                pltpu.async_copy(g_hbm.at[src_v.at[j + 2]], rows_a, sem_a)

            pltpu.make_async_copy(g_hbm.at[src_v.at[j + 1]], rows_b,
                                  sem_b).wait()
            pltpu.sync_copy(rows_b, acc_sh.at[dst_v.at[j + 1]], add=True)
            return 0

        lax.fori_loop(0, CH // 2, body, 0)
        plsc.subcore_barrier()

    with jax.named_scope("sc_writeback"):
        pltpu.sync_copy(acc_sh.at[pl.ds(row0, ROWS_PER_TILE)],
                        out_hbm.at[c, pl.ds(row0, ROWS_PER_TILE)])


def _scatter_call(g_pad, src2d, dst2d):
    return pl.kernel(
        _scatter_kernel,
        out_type=jax.ShapeDtypeStruct((NC, NP, H), jnp.float32),
        mesh=_sc_mesh(),
        compiler_params=pltpu.CompilerParams(needs_layout_passes=False),
        scratch_types=[
            pltpu.VMEM((CH, EDGE_CHUNK), jnp.int32),
            pltpu.VMEM((CH, EDGE_CHUNK), jnp.int32),
            pltpu.VMEM((EDGE_CHUNK, H), jnp.float32),
            pltpu.VMEM((EDGE_CHUNK, H), jnp.float32),
            pltpu.VMEM((16, H), jnp.float32),
            pltpu.VMEM_SHARED((NP, H), jnp.float32),
            pltpu.SemaphoreType.DMA,
            pltpu.SemaphoreType.DMA,
        ],
    )(g_pad, src2d, dst2d)


def _g_kernel(x_ref, w_ref, deg_ref, ones_ref, g_ref, dinv_ref):
    h = jnp.dot(x_ref[...], w_ref[...], preferred_element_type=jnp.float32)
    cnt = lax.dot_general(deg_ref[...], ones_ref[...],
                          (((0,), (0,)), ((), ())),
                          preferred_element_type=jnp.float32)
    dinv = lax.rsqrt(cnt + 1.0)
    g_ref[...] = h * dinv
    dinv_ref[...] = dinv


def _g_call(x, w_conv, deg_parts):
    blk = 1024
    grid = NP // blk
    ones = jnp.ones((NT, 1), jnp.float32)
    return pl.pallas_call(
        _g_kernel,
        grid=(grid,),
        in_specs=[
            pl.BlockSpec((blk, D_IN), lambda i: (i, 0)),
            pl.BlockSpec((D_IN, H), lambda i: (0, 0)),
            pl.BlockSpec((NT, blk), lambda i: (0, i)),
            pl.BlockSpec((NT, 1), lambda i: (0, 0)),
        ],
        out_specs=[
            pl.BlockSpec((blk, H), lambda i: (i, 0)),
            pl.BlockSpec((blk, 1), lambda i: (i, 0)),
        ],
        out_shape=[
            jax.ShapeDtypeStruct((NP, H), jnp.float32),
            jax.ShapeDtypeStruct((NP, 1), jnp.float32),
        ],
    )(x, w_conv, deg_parts, ones)


_DB = 512


def _pool_kernel(s0_ref, s1_ref, g_ref, dinv_ref, bconv_ref, batch_ref,
                 batch_s_ref, hp_ref, root_ref,
                 hsum_ref, hmax_ref, rootsum_ref, carry_ref):
    bi = pl.program_id(0)
    nb = pl.num_programs(0)

    @pl.when(bi == 0)
    def _init():
        hsum_ref[...] = jnp.zeros((G, H), jnp.float32)
        hmax_ref[...] = jnp.full((G, H), -jnp.inf, jnp.float32)
        rootsum_ref[...] = jnp.zeros((G,), jnp.int32)
        carry_ref[0] = batch_s_ref[0]
        carry_ref[1] = 0

    hc = (s0_ref[...] + s1_ref[...] + g_ref[...]) * dinv_ref[...]
    hc = jnp.maximum(hc + bconv_ref[...][None, :], 0.0)

    bb = batch_ref[...]
    ridx = lax.broadcasted_iota(jnp.int32, (_DB, 1), 0) + bi * _DB
    valid = ridx < N

    prev_last = carry_ref[0]
    cnt_carry = carry_ref[1]
    shifted = jnp.concatenate(
        [jnp.full((1, 1), prev_last, jnp.int32), bb[:-1]], axis=0)
    bvec = jnp.where((bb != shifted) & valid, 1, 0)

    k = bvec
    sh = 1
    while sh < _DB:
        k = k + jnp.concatenate(
            [jnp.zeros((sh, 1), jnp.int32), k[:-sh]], axis=0)
        sh *= 2
    k = k + cnt_carry

    kb = jnp.broadcast_to(k, (_DB, G))
    rows = lax.broadcasted_iota(jnp.int32, (_DB, G), 1)
    hit = (kb == rows) & jnp.broadcast_to(bvec == 1, (_DB, G))
    contrib = jnp.where(hit, jnp.broadcast_to(ridx, (_DB, G)), 0)
    rootsum_ref[...] += jnp.sum(contrib, axis=0)

    carry_ref[0] = batch_s_ref[_DB - 1]
    carry_ref[1] = cnt_carry + jnp.sum(bvec)

    gmin = batch_s_ref[0]
    gmax = batch_s_ref[_DB - 1]

    def gbody(gi, _):
        mask = (bb == gi) & valid
        ssum = jnp.sum(jnp.where(mask, hc, 0.0), axis=0, keepdims=True)
        smax = jnp.max(jnp.where(mask, hc, -jnp.inf), axis=0, keepdims=True)
        hsum_ref[pl.ds(gi, 1), :] += ssum
        hmax_ref[pl.ds(gi, 1), :] = jnp.maximum(hmax_ref[pl.ds(gi, 1), :],
                                                smax)
        return 0

    lax.fori_loop(gmin, gmax + 1, gbody, 0)

    @pl.when(bi == nb - 1)
    def _fin():
        hp_ref[...] = hmax_ref[...] + hsum_ref[...]
        iota_g = lax.broadcasted_iota(jnp.int32, (G,), 0)
        root_ref[...] = jnp.where(iota_g == 0, 0,
                                  jnp.maximum(rootsum_ref[...], 1))


def _pool_call(s0, s1, g_pad, dinv, b_conv, batch2d, batch_pad):
    grid = NP // _DB
    return pl.pallas_call(
        _pool_kernel,
        grid=(grid,),
        in_specs=[
            pl.BlockSpec((_DB, H), lambda i: (i, 0)),
            pl.BlockSpec((_DB, H), lambda i: (i, 0)),
            pl.BlockSpec((_DB, H), lambda i: (i, 0)),
            pl.BlockSpec((_DB, 1), lambda i: (i, 0)),
            pl.BlockSpec((H,), lambda i: (0,)),
            pl.BlockSpec((_DB, 1), lambda i: (i, 0)),
            pl.BlockSpec((_DB,), lambda i: (i,), memory_space=pltpu.SMEM),
        ],
        out_specs=[
            pl.BlockSpec((G, H), lambda i: (0, 0)),
            pl.BlockSpec((G,), lambda i: (0,)),
        ],
        out_shape=[
            jax.ShapeDtypeStruct((G, H), jnp.float32),
            jax.ShapeDtypeStruct((G,), jnp.int32),
        ],
        scratch_shapes=[
            pltpu.VMEM((G, H), jnp.float32),
            pltpu.VMEM((G, H), jnp.float32),
            pltpu.VMEM((G,), jnp.int32),
            pltpu.SMEM((2,), jnp.int32),
        ],
    )(s0, s1, g_pad, dinv, b_conv, batch2d, batch_pad)


def _head_kernel(x_ref, root_ref, hp_ref, w0_ref, b0_ref, w1_ref, b1_ref,
                 w2_ref, b2_ref, o_ref, xr_ref):
    def gbody(i, _):
        r = root_ref[i]
        xr_ref[pl.ds(i, 1), :] = x_ref[pl.ds(r, 1), :]
        return 0

    lax.fori_loop(0, G, gbody, 0)
    news = jnp.dot(xr_ref[...], w0_ref[...],
                   preferred_element_type=jnp.float32)
    news = jnp.maximum(news + b0_ref[...][None, :], 0.0)
    cat = jnp.concatenate([news, hp_ref[...]], axis=1)
    h2 = jnp.dot(cat, w1_ref[...], preferred_element_type=jnp.float32)
    h2 = jnp.maximum(h2 + b1_ref[...][None, :], 0.0)
    logits = jnp.dot(h2, w2_ref[...], preferred_element_type=jnp.float32)
    logits = logits + b2_ref[...][None, :]
    mx = jnp.max(logits, axis=1, keepdims=True)
    lse = jnp.log(jnp.sum(jnp.exp(logits - mx), axis=1, keepdims=True)) + mx
    o_ref[...] = logits - lse


def _head_call(x, root, hp, w0, b0, w1, b1, w2, b2):
    return pl.pallas_call(
        _head_kernel,
        in_specs=[
            pl.BlockSpec((N, D_IN), lambda: (0, 0)),
            pl.BlockSpec(memory_space=pltpu.SMEM),
            pl.BlockSpec((G, H), lambda: (0, 0)),
            pl.BlockSpec((D_IN, H), lambda: (0, 0)),
            pl.BlockSpec((H,), lambda: (0,)),
            pl.BlockSpec((2 * H, H), lambda: (0, 0)),
            pl.BlockSpec((H,), lambda: (0,)),
            pl.BlockSpec((H, C_OUT), lambda: (0, 0)),
            pl.BlockSpec((C_OUT,), lambda: (0,)),
        ],
        out_shape=jax.ShapeDtypeStruct((G, C_OUT), jnp.float32),
        scratch_shapes=[pltpu.VMEM((G, D_IN), jnp.float32)],
    )(x, root, hp, w0, b0, w1, b1, w2, b2)


def kernel(x, edge_index, batch, W_conv, b_conv, W0, b0, W1, b1, W2, b2):
    src = edge_index[0]
    dst = edge_index[1]
    pad = EP - E
    pad_rows = N + (jnp.arange(pad, dtype=jnp.int32) % (NP - N))
    src_p = jnp.concatenate([src, pad_rows])
    dst_p = jnp.concatenate([dst, pad_rows])
    src2d = src_p.reshape(EP // EDGE_CHUNK, EDGE_CHUNK)
    dst2d = dst_p.reshape(EP // EDGE_CHUNK, EDGE_CHUNK)

    batch_pad = jnp.concatenate(
        [batch, jnp.full((NP - N,), G - 1, jnp.int32)])

    deg_parts = _deg_call(dst_p)
    g_pad, dinv = _g_call(x, W_conv, deg_parts)
    s_parts = _scatter_call(g_pad, src2d, dst2d)
    hp, root = _pool_call(s_parts[0], s_parts[1], g_pad, dinv,
                          b_conv, batch_pad[:, None], batch_pad)
    return _head_call(x, root, hp, W0, b0, W1, b1, W2, b2)


# speedup vs baseline: 2.7645x; 1.0278x over previous
import functools

import jax
import jax.numpy as jnp
from jax import lax
from jax.experimental import pallas as pl
from jax.experimental.pallas import tpu as pltpu
from jax.experimental.pallas import tpu_sc as plsc

NC = 2
NS = 16
NT = NC * NS

N = 10000
NP = 10240
E = 160000
G = 64
D_IN = 256
H = 128
C_OUT = 2

EDGE_CHUNK = 128
CH = 40
EPT = CH * EDGE_CHUNK
EP = NT * EPT
ROWS_PER_TILE = NP // NS


def _sc_mesh():
    return plsc.VectorSubcoreMesh(core_axis_name="c", subcore_axis_name="s",
                                  num_cores=NC, num_subcores=NS)


def _deg_kernel(dst_hbm, out_hbm, idx_v, cnt_v):
    c = lax.axis_index("c")
    s = lax.axis_index("s")
    wid = c * NS + s

    zeros = jnp.zeros((16,), jnp.float32)

    def zero_body(i, _):
        cnt_v[pl.ds(i * 16, 16)] = zeros
        return 0

    lax.fori_loop(0, NP // 16, zero_body, 0)

    pltpu.sync_copy(dst_hbm.at[pl.ds(wid * EPT, EPT)], idx_v)

    ones = jnp.ones((16,), jnp.float32)

    def body(i, _):
        idx = idx_v[pl.ds(i * 16, 16)]
        plsc.addupdate_scatter(cnt_v, [idx], ones)
        return 0

    lax.fori_loop(0, EPT // 16, body, 0)
    pltpu.sync_copy(cnt_v, out_hbm.at[wid])


def _deg_call(dst_flat):
    return pl.kernel(
        _deg_kernel,
        out_type=jax.ShapeDtypeStruct((NT, NP), jnp.float32),
        mesh=_sc_mesh(),
        compiler_params=pltpu.CompilerParams(needs_layout_passes=False),
        scratch_types=[
            pltpu.VMEM((EPT,), jnp.int32),
            pltpu.VMEM((NP,), jnp.float32),
        ],
    )(dst_flat)


def _scatter_kernel(g_hbm, src_hbm, dst_hbm, out_hbm,
                    src_v, dst_v, rows_a, rows_b, zero_v, acc_sh,
                    sem_a, sem_b):
    c = lax.axis_index("c")
    s = lax.axis_index("s")
    wid = c * NS + s
    row0 = s * ROWS_PER_TILE

    zeros = jnp.zeros((16,), jnp.float32)

    with jax.named_scope("sc_zero"):
        def zbuf_body(i, _):
            for j in range(H // 16):
                zero_v[i, pl.ds(j * 16, 16)] = zeros
            return 0

        lax.fori_loop(0, 16, zbuf_body, 0)

        def zacc_body(i, _):
            pltpu.sync_copy(zero_v, acc_sh.at[pl.ds(row0 + i * 16, 16)])
            return 0

        lax.fori_loop(0, ROWS_PER_TILE // 16, zacc_body, 0)

    with jax.named_scope("sc_idx"):
        pltpu.sync_copy(src_hbm.at[pl.ds(wid * CH, CH)], src_v)
        pltpu.sync_copy(dst_hbm.at[pl.ds(wid * CH, CH)], dst_v)
        plsc.subcore_barrier()

    with jax.named_scope("sc_edges"):
        pltpu.async_copy(g_hbm.at[src_v.at[0]], rows_a, sem_a)

        def body(t, _):
            j = 2 * t
            pltpu.async_copy(g_hbm.at[src_v.at[j + 1]], rows_b, sem_b)
            pltpu.make_async_copy(g_hbm.at[src_v.at[j]], rows_a, sem_a).wait()
            pltpu.sync_copy(rows_a, acc_sh.at[dst_v.at[j]], add=True)

            @pl.when(t + 1 < CH // 2)
            def _():
                pltpu.async_copy(g_hbm.at[src_v.at[j + 2]], rows_a, sem_a)

            pltpu.make_async_copy(g_hbm.at[src_v.at[j + 1]], rows_b,
                                  sem_b).wait()
            pltpu.sync_copy(rows_b, acc_sh.at[dst_v.at[j + 1]], add=True)
            return 0

        lax.fori_loop(0, CH // 2, body, 0)
        plsc.subcore_barrier()

    with jax.named_scope("sc_writeback"):
        pltpu.sync_copy(acc_sh.at[pl.ds(row0, ROWS_PER_TILE)],
                        out_hbm.at[c, pl.ds(row0, ROWS_PER_TILE)])


def _scatter_call(g_pad, src2d, dst2d):
    return pl.kernel(
        _scatter_kernel,
        out_type=jax.ShapeDtypeStruct((NC, NP, H), jnp.float32),
        mesh=_sc_mesh(),
        compiler_params=pltpu.CompilerParams(needs_layout_passes=False),
        scratch_types=[
            pltpu.VMEM((CH, EDGE_CHUNK), jnp.int32),
            pltpu.VMEM((CH, EDGE_CHUNK), jnp.int32),
            pltpu.VMEM((EDGE_CHUNK, H), jnp.float32),
            pltpu.VMEM((EDGE_CHUNK, H), jnp.float32),
            pltpu.VMEM((16, H), jnp.float32),
            pltpu.VMEM_SHARED((NP, H), jnp.float32),
            pltpu.SemaphoreType.DMA,
            pltpu.SemaphoreType.DMA,
        ],
    )(g_pad, src2d, dst2d)


def _g_kernel(x_ref, w_ref, deg_ref, ones_ref, g_ref, dinv_ref):
    h = jnp.dot(x_ref[...], w_ref[...], preferred_element_type=jnp.float32)
    cnt = lax.dot_general(deg_ref[...], ones_ref[...],
                          (((0,), (0,)), ((), ())),
                          preferred_element_type=jnp.float32)
    dinv = lax.rsqrt(cnt + 1.0)
    g_ref[...] = h * dinv
    dinv_ref[...] = dinv


def _g_call(x, w_conv, deg_parts):
    blk = 1024
    grid = NP // blk
    ones = jnp.ones((NT, 1), jnp.float32)
    return pl.pallas_call(
        _g_kernel,
        grid=(grid,),
        in_specs=[
            pl.BlockSpec((blk, D_IN), lambda i: (i, 0)),
            pl.BlockSpec((D_IN, H), lambda i: (0, 0)),
            pl.BlockSpec((NT, blk), lambda i: (0, i)),
            pl.BlockSpec((NT, 1), lambda i: (0, 0)),
        ],
        out_specs=[
            pl.BlockSpec((blk, H), lambda i: (i, 0)),
            pl.BlockSpec((blk, 1), lambda i: (i, 0)),
        ],
        out_shape=[
            jax.ShapeDtypeStruct((NP, H), jnp.float32),
            jax.ShapeDtypeStruct((NP, 1), jnp.float32),
        ],
    )(x, w_conv, deg_parts, ones)


_DB = 512


def _pool_kernel(s0_ref, s1_ref, g_ref, dinv_ref, bconv_ref, batch_ref,
                 batch_s_ref, hp_ref, root_ref,
                 hsum_ref, hmax_ref, rootsum_ref, carry_ref):
    bi = pl.program_id(0)
    nb = pl.num_programs(0)

    @pl.when(bi == 0)
    def _init():
        hsum_ref[...] = jnp.zeros((G, H), jnp.float32)
        hmax_ref[...] = jnp.full((G, H), -jnp.inf, jnp.float32)
        rootsum_ref[...] = jnp.zeros((G,), jnp.int32)
        carry_ref[0] = batch_s_ref[0]
        carry_ref[1] = 0

    bb = batch_ref[...]
    ridx = lax.broadcasted_iota(jnp.int32, (_DB, 1), 0) + bi * _DB
    valid = ridx < N

    hc = (s0_ref[0] + s1_ref[0] + g_ref[...]) * dinv_ref[...]
    hc = jnp.maximum(hc + bconv_ref[...][None, :], 0.0)
    hc = jnp.where(valid, hc, 0.0)

    prev_last = carry_ref[0]
    cnt_carry = carry_ref[1]
    shifted = jnp.concatenate(
        [jnp.full((1, 1), prev_last, jnp.int32), bb[:-1]], axis=0)
    bvec = jnp.where((bb != shifted) & valid, 1, 0)

    k = bvec
    sh = 1
    while sh < _DB:
        k = k + jnp.concatenate(
            [jnp.zeros((sh, 1), jnp.int32), k[:-sh]], axis=0)
        sh *= 2
    k = k + cnt_carry

    kb = jnp.broadcast_to(k, (_DB, G))
    rows = lax.broadcasted_iota(jnp.int32, (_DB, G), 1)
    hit = (kb == rows) & jnp.broadcast_to(bvec == 1, (_DB, G))
    contrib = jnp.where(hit, jnp.broadcast_to(ridx, (_DB, G)), 0)
    rootsum_ref[...] += jnp.sum(contrib, axis=0)

    carry_ref[0] = batch_s_ref[_DB - 1]
    carry_ref[1] = cnt_carry + jnp.sum(bvec)

    onehot = ((jnp.broadcast_to(bb, (_DB, G)) == rows) &
              jnp.broadcast_to(valid, (_DB, G))).astype(jnp.float32)
    hsum_ref[...] += lax.dot_general(onehot, hc, (((0,), (0,)), ((), ())),
                                     preferred_element_type=jnp.float32)

    gmin = batch_s_ref[0]
    gmax = batch_s_ref[_DB - 1]

    def gbody(gi, _):
        mask = (bb == gi) & valid
        smax = jnp.max(jnp.where(mask, hc, -jnp.inf), axis=0, keepdims=True)
        hmax_ref[pl.ds(gi, 1), :] = jnp.maximum(hmax_ref[pl.ds(gi, 1), :],
                                                smax)
        return 0

    lax.fori_loop(gmin, gmax + 1, gbody, 0)

    @pl.when(bi == nb - 1)
    def _fin():
        hp_ref[...] = hmax_ref[...] + hsum_ref[...]
        iota_g = lax.broadcasted_iota(jnp.int32, (G,), 0)
        root_ref[...] = jnp.where(iota_g == 0, 0,
                                  jnp.maximum(rootsum_ref[...], 1))


def _pool_call(s_parts, g_pad, dinv, b_conv, batch2d, batch_pad):
    grid = NP // _DB
    return pl.pallas_call(
        _pool_kernel,
        grid=(grid,),
        in_specs=[
            pl.BlockSpec((1, _DB, H), lambda i: (0, i, 0)),
            pl.BlockSpec((1, _DB, H), lambda i: (1, i, 0)),
            pl.BlockSpec((_DB, H), lambda i: (i, 0)),
            pl.BlockSpec((_DB, 1), lambda i: (i, 0)),
            pl.BlockSpec((H,), lambda i: (0,)),
            pl.BlockSpec((_DB, 1), lambda i: (i, 0)),
            pl.BlockSpec((_DB,), lambda i: (i,), memory_space=pltpu.SMEM),
        ],
        out_specs=[
            pl.BlockSpec((G, H), lambda i: (0, 0)),
            pl.BlockSpec((G,), lambda i: (0,)),
        ],
        out_shape=[
            jax.ShapeDtypeStruct((G, H), jnp.float32),
            jax.ShapeDtypeStruct((G,), jnp.int32),
        ],
        scratch_shapes=[
            pltpu.VMEM((G, H), jnp.float32),
            pltpu.VMEM((G, H), jnp.float32),
            pltpu.VMEM((G,), jnp.int32),
            pltpu.SMEM((2,), jnp.int32),
        ],
    )(s_parts, s_parts, g_pad, dinv, b_conv, batch2d, batch_pad)


def _head_kernel(x_ref, root_ref, hp_ref, w0_ref, b0_ref, w1_ref, b1_ref,
                 w2_ref, b2_ref, o_ref, xr_ref):
    def gbody(i, _):
        r = root_ref[i]
        xr_ref[pl.ds(i, 1), :] = x_ref[pl.ds(r, 1), :]
        return 0

    lax.fori_loop(0, G, gbody, 0)
    news = jnp.dot(xr_ref[...], w0_ref[...],
                   preferred_element_type=jnp.float32)
    news = jnp.maximum(news + b0_ref[...][None, :], 0.0)
    cat = jnp.concatenate([news, hp_ref[...]], axis=1)
    h2 = jnp.dot(cat, w1_ref[...], preferred_element_type=jnp.float32)
    h2 = jnp.maximum(h2 + b1_ref[...][None, :], 0.0)
    logits = jnp.dot(h2, w2_ref[...], preferred_element_type=jnp.float32)
    logits = logits + b2_ref[...][None, :]
    mx = jnp.max(logits, axis=1, keepdims=True)
    lse = jnp.log(jnp.sum(jnp.exp(logits - mx), axis=1, keepdims=True)) + mx
    o_ref[...] = logits - lse


def _head_call(x, root, hp, w0, b0, w1, b1, w2, b2):
    return pl.pallas_call(
        _head_kernel,
        in_specs=[
            pl.BlockSpec((N, D_IN), lambda: (0, 0)),
            pl.BlockSpec(memory_space=pltpu.SMEM),
            pl.BlockSpec((G, H), lambda: (0, 0)),
            pl.BlockSpec((D_IN, H), lambda: (0, 0)),
            pl.BlockSpec((H,), lambda: (0,)),
            pl.BlockSpec((2 * H, H), lambda: (0, 0)),
            pl.BlockSpec((H,), lambda: (0,)),
            pl.BlockSpec((H, C_OUT), lambda: (0, 0)),
            pl.BlockSpec((C_OUT,), lambda: (0,)),
        ],
        out_shape=jax.ShapeDtypeStruct((G, C_OUT), jnp.float32),
        scratch_shapes=[pltpu.VMEM((G, D_IN), jnp.float32)],
    )(x, root, hp, w0, b0, w1, b1, w2, b2)


def kernel(x, edge_index, batch, W_conv, b_conv, W0, b0, W1, b1, W2, b2):
    src = edge_index[0]
    dst = edge_index[1]
    pad = EP - E
    pad_rows = N + (jnp.arange(pad, dtype=jnp.int32) % (NP - N))
    src_p = jnp.concatenate([src, pad_rows])
    dst_p = jnp.concatenate([dst, pad_rows])
    src2d = src_p.reshape(EP // EDGE_CHUNK, EDGE_CHUNK)
    dst2d = dst_p.reshape(EP // EDGE_CHUNK, EDGE_CHUNK)

    batch_pad = jnp.concatenate(
        [batch, jnp.full((NP - N,), G - 1, jnp.int32)])

    deg_parts = _deg_call(dst_p)
    g_pad, dinv = _g_call(x, W_conv, deg_parts)
    s_parts = _scatter_call(g_pad, src2d, dst2d)
    hp, root = _pool_call(s_parts, g_pad, dinv,
                          b_conv, batch_pad[:, None], batch_pad)
    return _head_call(x, root, hp, W0, b0, W1, b1, W2, b2)
